# baseline (speedup 1.0000x reference)
import functools

import jax
import jax.numpy as jnp
from jax.experimental import pallas as pl
from jax.experimental.pallas import tpu as pltpu

_TOPK = 3


def _fused_body(gw_r_ref, gw_i_ref, pm_r_ref, pm_i_ref, Wg_ref, bg_ref,
                Wa_ref, ba_ref, gr_ref, br_ref, gi_ref, bi_ref,
                read_ref, next_ref, ent_ref, *, total_b):
    gw_r = gw_r_ref[...]
    gw_i = gw_i_ref[...]
    pm_r = pm_r_ref[...]
    pm_i = pm_i_ref[...]
    bB, S, D = pm_r.shape

    sim = jnp.sum(pm_r * gw_r[:, None, :] + pm_i * gw_i[:, None, :], axis=-1)
    sim = sim - jnp.max(sim, axis=-1, keepdims=True)
    es = jnp.exp(sim)
    attn = es / jnp.sum(es, axis=-1, keepdims=True)
    read_ref[0] = jnp.sum(pm_r * attn[:, :, None], axis=1)
    read_ref[1] = jnp.sum(pm_i * attn[:, :, None], axis=1)

    flat = jnp.concatenate([gw_r, gw_i], axis=-1)
    gate_logit = jnp.sum(flat * Wg_ref[...], axis=-1, keepdims=True) + bg_ref[0, 0]
    write_gate = jax.nn.sigmoid(gate_logit)

    logits = jax.lax.dot_general(flat, Wa_ref[...], (((1,), (1,)), ((), ())),
                                 preferred_element_type=jnp.float32)
    logits = logits + ba_ref[...]
    logits = logits - jnp.max(logits, axis=-1, keepdims=True)
    ew = jnp.exp(logits)
    w = ew / jnp.sum(ew, axis=-1, keepdims=True)

    ent_rows = -jnp.sum(w * jnp.log(w + 1e-10), axis=-1, keepdims=True)
    ent_part = jnp.sum(ent_rows, axis=0, keepdims=True) / total_b
    i = pl.program_id(0)

    @pl.when(i == 0)
    def _():
        ent_ref[...] = ent_part

    @pl.when(i != 0)
    def _():
        ent_ref[...] += ent_part

    iota = jax.lax.broadcasted_iota(jnp.int32, (bB, S), 1)
    remaining = w
    keep = jnp.zeros(w.shape, dtype=jnp.bool_)
    for _ in range(_TOPK):
        m = jnp.max(remaining, axis=-1, keepdims=True)
        first = jnp.min(jnp.where(remaining == m, iota, S), axis=-1,
                        keepdims=True)
        onehot = iota == first
        keep = jnp.logical_or(keep, onehot)
        remaining = jnp.where(onehot, -1.0, remaining)
    sparse = jnp.where(keep, w, 0.0)
    sparse = sparse / (jnp.sum(sparse, axis=-1, keepdims=True) + 1e-6)

    eff = (write_gate * sparse)[:, :, None]

    def _update(pm, gw, gamma, beta):
        x = jnp.tanh(pm + eff * (gw[:, None, :] - pm))
        mu = jnp.mean(x, axis=-1, keepdims=True)
        var = jnp.mean((x - mu) ** 2, axis=-1, keepdims=True)
        return (x - mu) * jax.lax.rsqrt(var + 1e-6) * gamma[None] + beta[None]

    next_ref[0] = _update(pm_r, gw_r, gr_ref[...], br_ref[...])
    next_ref[1] = _update(pm_i, gw_i, gi_ref[...], bi_ref[...])


def kernel(gw_state_real, gw_state_imag, prev_mem_real, prev_mem_imag,
           Wg, bg, Wa, ba, gamma_r, beta_r, gamma_i, beta_i):
    B, S, D = prev_mem_real.shape
    bB = 64
    grid = (B // bB,)

    bg2 = bg.reshape(1, 1)
    ba2 = ba.reshape(1, S)
    gr2 = gamma_r.reshape(1, D)
    br2 = beta_r.reshape(1, D)
    gi2 = gamma_i.reshape(1, D)
    bi2 = beta_i.reshape(1, D)

    def row_map(i):
        return (i, 0)

    def mem_map(i):
        return (i, 0, 0)

    def const2(i):
        return (0, 0)

    read_out, next_mem, ent = pl.pallas_call(
        functools.partial(_fused_body, total_b=float(B)),
        grid=grid,
        in_specs=[
            pl.BlockSpec((bB, D), row_map),
            pl.BlockSpec((bB, D), row_map),
            pl.BlockSpec((bB, S, D), mem_map),
            pl.BlockSpec((bB, S, D), mem_map),
            pl.BlockSpec((1, 2 * D), const2),
            pl.BlockSpec((1, 1), const2),
            pl.BlockSpec((S, 2 * D), const2),
            pl.BlockSpec((1, S), const2),
            pl.BlockSpec((1, D), const2),
            pl.BlockSpec((1, D), const2),
            pl.BlockSpec((1, D), const2),
            pl.BlockSpec((1, D), const2),
        ],
        out_specs=[
            pl.BlockSpec((2, bB, D), lambda i: (0, i, 0)),
            pl.BlockSpec((2, bB, S, D), lambda i: (0, i, 0, 0)),
            pl.BlockSpec((1, 1), const2),
        ],
        out_shape=[
            jax.ShapeDtypeStruct((2, B, D), jnp.float32),
            jax.ShapeDtypeStruct((2, B, S, D), jnp.float32),
            jax.ShapeDtypeStruct((1, 1), jnp.float32),
        ],
    )(gw_state_real, gw_state_imag, prev_mem_real, prev_mem_imag,
      Wg, bg2, Wa, ba2, gr2, br2, gi2, bi2)

    return (read_out, next_mem, ent[0, 0])
